# baseline (speedup 1.0000x reference)
import jax
import jax.numpy as jnp
from jax.experimental import pallas as pl

T, D_MODEL, D_FF, E, TOP_K = 32, 768, 1536, 8, 2
BF = 512
NF = D_FF // BF


def _moe_body(x_ref, ids_ref, tw_ref, w13_ref, w2_ref, out_ref):
    e = pl.program_id(0)
    f = pl.program_id(1)

    @pl.when((e == 0) & (f == 0))
    def _init():
        out_ref[...] = jnp.zeros_like(out_ref)

    x = x_ref[...]
    w1 = w13_ref[0, 0]
    w3 = w13_ref[0, 1]
    h1 = jax.lax.dot_general(
        x, w1, (((1,), (1,)), ((), ())),
        preferred_element_type=jnp.float32)
    h3 = jax.lax.dot_general(
        x, w3, (((1,), (1,)), ((), ())),
        preferred_element_type=jnp.float32)
    act = h1 * jax.nn.sigmoid(h1) * h3
    o = jax.lax.dot_general(
        act, w2_ref[0], (((1,), (1,)), ((), ())),
        preferred_element_type=jnp.float32)

    gate = jnp.sum(
        jnp.where(ids_ref[...] == e, tw_ref[...], 0.0),
        axis=1, keepdims=True)
    out_ref[...] += gate * o


@jax.jit
def kernel(x, topk_ids, topk_weight, w13_weight, w2_weight):
    w13 = w13_weight.reshape(E, 2, D_FF, D_MODEL)
    return pl.pallas_call(
        _moe_body,
        grid=(E, NF),
        in_specs=[
            pl.BlockSpec((T, D_MODEL), lambda e, f: (0, 0)),
            pl.BlockSpec((T, TOP_K), lambda e, f: (0, 0)),
            pl.BlockSpec((T, TOP_K), lambda e, f: (0, 0)),
            pl.BlockSpec((1, 2, BF, D_MODEL), lambda e, f: (e, 0, f, 0)),
            pl.BlockSpec((1, D_MODEL, BF), lambda e, f: (e, 0, f)),
        ],
        out_specs=pl.BlockSpec((T, D_MODEL), lambda e, f: (0, 0)),
        out_shape=jax.ShapeDtypeStruct((T, D_MODEL), jnp.float32),
    )(x, topk_ids, topk_weight, w13, w2_weight)


# speedup vs baseline: 1.1295x; 1.1295x over previous
import jax
import jax.numpy as jnp
from jax.experimental import pallas as pl

T, D_MODEL, D_FF, E, TOP_K = 32, 768, 1536, 8, 2
HM = D_MODEL // 2


def _moe_body(x_ref, ids_ref, tw_ref, w1_ref, w3_ref, w2a_ref, w2b_ref,
              out_ref):
    e = pl.program_id(0)

    @pl.when(e == 0)
    def _init():
        out_ref[...] = jnp.zeros_like(out_ref)

    x = x_ref[...]
    h1 = jax.lax.dot_general(
        x, w1_ref[0, 0], (((1,), (1,)), ((), ())),
        preferred_element_type=jnp.float32)
    h3 = jax.lax.dot_general(
        x, w3_ref[0, 0], (((1,), (1,)), ((), ())),
        preferred_element_type=jnp.float32)
    act = h1 * jax.nn.sigmoid(h1) * h3
    oa = jax.lax.dot_general(
        act, w2a_ref[0, 0], (((1,), (1,)), ((), ())),
        preferred_element_type=jnp.float32)
    ob = jax.lax.dot_general(
        act, w2b_ref[0, 0], (((1,), (1,)), ((), ())),
        preferred_element_type=jnp.float32)

    gate = jnp.sum(
        jnp.where(ids_ref[...] == e, tw_ref[...], 0.0),
        axis=1, keepdims=True)
    out_ref[:, :HM] += gate * oa
    out_ref[:, HM:] += gate * ob


@jax.jit
def kernel(x, topk_ids, topk_weight, w13_weight, w2_weight):
    w13 = w13_weight.reshape(E, 2, D_FF, D_MODEL)
    w2 = w2_weight.reshape(E, 2, HM, D_FF)
    return pl.pallas_call(
        _moe_body,
        grid=(E,),
        in_specs=[
            pl.BlockSpec((T, D_MODEL), lambda e: (0, 0)),
            pl.BlockSpec((T, TOP_K), lambda e: (0, 0)),
            pl.BlockSpec((T, TOP_K), lambda e: (0, 0)),
            pl.BlockSpec((1, 1, D_FF, D_MODEL), lambda e: (e, 0, 0, 0)),
            pl.BlockSpec((1, 1, D_FF, D_MODEL), lambda e: (e, 1, 0, 0)),
            pl.BlockSpec((1, 1, HM, D_FF), lambda e: (e, 0, 0, 0)),
            pl.BlockSpec((1, 1, HM, D_FF), lambda e: (e, 1, 0, 0)),
        ],
        out_specs=pl.BlockSpec((T, D_MODEL), lambda e: (0, 0)),
        out_shape=jax.ShapeDtypeStruct((T, D_MODEL), jnp.float32),
    )(x, topk_ids, topk_weight, w13, w13, w2, w2)
